# Initial kernel scaffold; baseline (speedup 1.0000x reference)
import jax
import jax.numpy as jnp
from jax.experimental import pallas as pl


def kernel(visit_rep, E_mem_patient_rep, E_mem_med_rep, Wq, bq, Wk, bk, Wv, bv, Wo, bo, W1, b1, W2, b2, g1, be1, g2, be2):
    raise NotImplementedError("write your pallas kernel here")


# speedup vs baseline: 10.1007x; 10.1007x over previous
import math
import jax
import jax.numpy as jnp
from jax.experimental import pallas as pl
from jax.experimental.pallas import tpu as pltpu

N = 2048
M = 1024
D = 1024
H = 16
DH = D // H
TOP_N = 10

_BN_ATTN = 256
_BN_FFN = 512


def _proj_kernel(x_ref, w_ref, b_ref, o_ref):
    o_ref[...] = jax.lax.dot_general(
        x_ref[...], w_ref[...], (((1,), (1,)), ((), ())),
        preferred_element_type=jnp.float32) + b_ref[...]


def _proj(x, w, b):
    n = x.shape[0]
    bn = 512
    return pl.pallas_call(
        _proj_kernel,
        grid=(n // bn,),
        in_specs=[
            pl.BlockSpec((bn, D), lambda i: (i, 0)),
            pl.BlockSpec((D, D), lambda i: (0, 0)),
            pl.BlockSpec((1, D), lambda i: (0, 0)),
        ],
        out_specs=pl.BlockSpec((bn, D), lambda i: (i, 0)),
        out_shape=jax.ShapeDtypeStruct((n, D), jnp.float32),
    )(x, w, b.reshape(1, D))


def _attn_kernel(q_ref, k_ref, v_ref, o_ref):
    q = q_ref[0]
    k = k_ref[0]
    v = v_ref[0]
    s = jax.lax.dot_general(
        q, k, (((1,), (1,)), ((), ())),
        preferred_element_type=jnp.float32) * (1.0 / math.sqrt(DH))
    neg = jnp.float32(-jnp.inf)
    prev = jnp.full((s.shape[0], 1), jnp.inf, jnp.float32)
    cnt = jnp.zeros((s.shape[0], 1), jnp.float32)
    thr = jnp.full((s.shape[0], 1), neg, jnp.float32)
    for _ in range(TOP_N):
        cand = jnp.where(s < prev, s, neg)
        m = jnp.max(cand, axis=1, keepdims=True)
        c = jnp.sum(jnp.where(s == m, 1.0, 0.0), axis=1, keepdims=True)
        active = cnt < TOP_N
        thr = jnp.where(active, m, thr)
        cnt = cnt + jnp.where(active, c, 0.0)
        prev = jnp.where(active, m, prev)
    keep = s >= thr
    rowmax = jnp.max(s, axis=1, keepdims=True)
    p = jnp.where(keep, jnp.exp(s - rowmax), 0.0)
    p = p / jnp.sum(p, axis=1, keepdims=True)
    o_ref[0] = jnp.dot(p, v, preferred_element_type=jnp.float32)


def _attention(qh, kh, vh):
    return pl.pallas_call(
        _attn_kernel,
        grid=(H, N // _BN_ATTN),
        in_specs=[
            pl.BlockSpec((1, _BN_ATTN, DH), lambda h, i: (h, i, 0)),
            pl.BlockSpec((1, M, DH), lambda h, i: (h, 0, 0)),
            pl.BlockSpec((1, M, DH), lambda h, i: (h, 0, 0)),
        ],
        out_specs=pl.BlockSpec((1, _BN_ATTN, DH), lambda h, i: (h, i, 0)),
        out_shape=jax.ShapeDtypeStruct((H, N, DH), jnp.float32),
    )(qh, kh, vh)


def _layernorm(x, g, b):
    mu = jnp.mean(x, axis=-1, keepdims=True)
    var = jnp.mean((x - mu) ** 2, axis=-1, keepdims=True)
    return (x - mu) * jax.lax.rsqrt(var + 1e-5) * g + b


def _tail_kernel(x_ref, a_ref, wo_ref, bo_ref, w1_ref, b1_ref, w2_ref,
                 b2_ref, g1_ref, be1_ref, g2_ref, be2_ref, o_ref):
    x = x_ref[...]
    z = jax.lax.dot_general(
        a_ref[...], wo_ref[...], (((1,), (1,)), ((), ())),
        preferred_element_type=jnp.float32) + bo_ref[...]
    x1 = _layernorm(x + z, g1_ref[...], be1_ref[...])
    h1 = jax.lax.dot_general(
        x1, w1_ref[...], (((1,), (1,)), ((), ())),
        preferred_element_type=jnp.float32) + b1_ref[...]
    h1 = jnp.where(h1 >= 0.0, h1, 0.01 * h1)
    ff = jax.lax.dot_general(
        h1, w2_ref[...], (((1,), (1,)), ((), ())),
        preferred_element_type=jnp.float32) + b2_ref[...]
    o_ref[...] = _layernorm(x1 + ff, g2_ref[...], be2_ref[...])


def _tail(x, attn_flat, Wo, bo, W1, b1, W2, b2, g1, be1, g2, be2):
    row = lambda t: t.reshape(1, D)
    full = pl.BlockSpec((D, D), lambda i: (0, 0))
    vec = pl.BlockSpec((1, D), lambda i: (0, 0))
    blk = pl.BlockSpec((_BN_FFN, D), lambda i: (i, 0))
    return pl.pallas_call(
        _tail_kernel,
        grid=(N // _BN_FFN,),
        in_specs=[blk, blk, full, vec, full, vec, full, vec,
                  vec, vec, vec, vec],
        out_specs=blk,
        out_shape=jax.ShapeDtypeStruct((N, D), jnp.float32),
    )(x, attn_flat, Wo, row(bo), W1, row(b1), W2, row(b2),
      row(g1), row(be1), row(g2), row(be2))


def kernel(visit_rep, E_mem_patient_rep, E_mem_med_rep, Wq, bq, Wk, bk,
           Wv, bv, Wo, bo, W1, b1, W2, b2, g1, be1, g2, be2):
    q = _proj(visit_rep, Wq, bq)
    k = _proj(E_mem_patient_rep, Wk, bk)
    v = _proj(E_mem_med_rep, Wv, bv)
    qh = q.reshape(N, H, DH).transpose(1, 0, 2)
    kh = k.reshape(M, H, DH).transpose(1, 0, 2)
    vh = v.reshape(M, H, DH).transpose(1, 0, 2)
    attn = _attention(qh, kh, vh)
    attn_flat = attn.transpose(1, 0, 2).reshape(N, D)
    return _tail(visit_rep, attn_flat, Wo, bo, W1, b1, W2, b2,
                 g1, be1, g2, be2)


# speedup vs baseline: 14.1246x; 1.3984x over previous
import math
import jax
import jax.numpy as jnp
from jax.experimental import pallas as pl
from jax.experimental.pallas import tpu as pltpu

N = 2048
M = 1024
D = 1024
H = 16
DH = D // H
TOP_N = 10

_BN = 512
_BN_FFN = 512


def _mm_t(a, b):
    return jax.lax.dot_general(
        a, b, (((1,), (1,)), ((), ())), preferred_element_type=jnp.float32)


def _kv_kernel(ep_ref, em_ref, wk_ref, bk_ref, wv_ref, bv_ref,
               k_ref, v_ref):
    kf = _mm_t(ep_ref[...], wk_ref[...]) + bk_ref[...]
    vf = _mm_t(em_ref[...], wv_ref[...]) + bv_ref[...]
    for hh in range(H):
        k_ref[hh] = kf[:, hh * DH:(hh + 1) * DH]
        v_ref[hh] = vf[:, hh * DH:(hh + 1) * DH]


def _kv(ep, em, Wk, bk, Wv, bv):
    row = lambda t: t.reshape(1, D)
    bm = 512
    full = pl.BlockSpec((D, D), lambda i: (0, 0))
    vec = pl.BlockSpec((1, D), lambda i: (0, 0))
    return pl.pallas_call(
        _kv_kernel,
        grid=(M // bm,),
        in_specs=[
            pl.BlockSpec((bm, D), lambda i: (i, 0)),
            pl.BlockSpec((bm, D), lambda i: (i, 0)),
            full, vec, full, vec,
        ],
        out_specs=[
            pl.BlockSpec((H, bm, DH), lambda i: (0, i, 0)),
            pl.BlockSpec((H, bm, DH), lambda i: (0, i, 0)),
        ],
        out_shape=[
            jax.ShapeDtypeStruct((H, M, DH), jnp.float32),
            jax.ShapeDtypeStruct((H, M, DH), jnp.float32),
        ],
    )(ep, em, Wk, row(bk), Wv, row(bv))


def _attn_kernel(x_ref, kh_ref, vh_ref, wq_ref, bq_ref, o_ref, q_s):
    h = pl.program_id(1)

    @pl.when(h == 0)
    def _init_q():
        qf = _mm_t(x_ref[...], wq_ref[...]) + bq_ref[...]
        for hh in range(H):
            q_s[hh] = qf[:, hh * DH:(hh + 1) * DH]

    outs = []
    for sub in range(2):
        hh = 2 * h + sub
        q = q_s[hh]
        k = kh_ref[hh]
        v = vh_ref[hh]
        s = _mm_t(q, k) * (1.0 / math.sqrt(DH))

        neg = jnp.float32(-jnp.inf)
        cnt = jnp.zeros((s.shape[0], 1), jnp.float32)
        thr = jnp.full((s.shape[0], 1), neg, jnp.float32)
        s_m = s
        for _ in range(TOP_N):
            m = jnp.max(s_m, axis=1, keepdims=True)
            mask = s_m == m
            c = jnp.sum(jnp.where(mask, 1.0, 0.0), axis=1, keepdims=True)
            s_m = jnp.where(mask, neg, s_m)
            active = cnt < TOP_N
            thr = jnp.where(active, m, thr)
            cnt = cnt + jnp.where(active, c, 0.0)
        keep = s >= thr
        rowmax = jnp.max(s, axis=1, keepdims=True)
        p = jnp.where(keep, jnp.exp(s - rowmax), 0.0)
        p = p / jnp.sum(p, axis=1, keepdims=True)
        outs.append(jnp.dot(p, v, preferred_element_type=jnp.float32))
    o_ref[...] = jnp.concatenate(outs, axis=1)


def _attention(x, kh, vh, Wq, bq):
    row = lambda t: t.reshape(1, D)
    return pl.pallas_call(
        _attn_kernel,
        grid=(N // _BN, H // 2),
        in_specs=[
            pl.BlockSpec((_BN, D), lambda i, h: (i, 0)),
            pl.BlockSpec((H, M, DH), lambda i, h: (0, 0, 0)),
            pl.BlockSpec((H, M, DH), lambda i, h: (0, 0, 0)),
            pl.BlockSpec((D, D), lambda i, h: (0, 0)),
            pl.BlockSpec((1, D), lambda i, h: (0, 0)),
        ],
        out_specs=pl.BlockSpec((_BN, 2 * DH), lambda i, h: (i, h)),
        out_shape=jax.ShapeDtypeStruct((N, D), jnp.float32),
        scratch_shapes=[
            pltpu.VMEM((H, _BN, DH), jnp.float32),
        ],
    )(x, kh, vh, Wq, row(bq))


def _layernorm(x, g, b):
    mu = jnp.mean(x, axis=-1, keepdims=True)
    var = jnp.mean((x - mu) ** 2, axis=-1, keepdims=True)
    return (x - mu) * jax.lax.rsqrt(var + 1e-5) * g + b


def _tail_kernel(x_ref, a_ref, wo_ref, bo_ref, w1_ref, b1_ref, w2_ref,
                 b2_ref, g1_ref, be1_ref, g2_ref, be2_ref, o_ref):
    x = x_ref[...]
    z = _mm_t(a_ref[...], wo_ref[...]) + bo_ref[...]
    x1 = _layernorm(x + z, g1_ref[...], be1_ref[...])
    h1 = _mm_t(x1, w1_ref[...]) + b1_ref[...]
    h1 = jnp.where(h1 >= 0.0, h1, 0.01 * h1)
    ff = _mm_t(h1, w2_ref[...]) + b2_ref[...]
    o_ref[...] = _layernorm(x1 + ff, g2_ref[...], be2_ref[...])


def _tail(x, attn_flat, Wo, bo, W1, b1, W2, b2, g1, be1, g2, be2):
    row = lambda t: t.reshape(1, D)
    full = pl.BlockSpec((D, D), lambda i: (0, 0))
    vec = pl.BlockSpec((1, D), lambda i: (0, 0))
    blk = pl.BlockSpec((_BN_FFN, D), lambda i: (i, 0))
    return pl.pallas_call(
        _tail_kernel,
        grid=(N // _BN_FFN,),
        in_specs=[blk, blk, full, vec, full, vec, full, vec,
                  vec, vec, vec, vec],
        out_specs=blk,
        out_shape=jax.ShapeDtypeStruct((N, D), jnp.float32),
    )(x, attn_flat, Wo, row(bo), W1, row(b1), W2, row(b2),
      row(g1), row(be1), row(g2), row(be2))


def kernel(visit_rep, E_mem_patient_rep, E_mem_med_rep, Wq, bq, Wk, bk,
           Wv, bv, Wo, bo, W1, b1, W2, b2, g1, be1, g2, be2):
    kh, vh = _kv(E_mem_patient_rep, E_mem_med_rep, Wk, bk, Wv, bv)
    attn_flat = _attention(visit_rep, kh, vh, Wq, bq)
    return _tail(visit_rep, attn_flat, Wo, bo, W1, b1, W2, b2,
                 g1, be1, g2, be2)
